# baseline (speedup 1.0000x reference)
import jax
import jax.numpy as jnp
from jax import lax
from jax.experimental import pallas as pl
from jax.experimental.pallas import tpu as pltpu
from jax.experimental.pallas import tpu_sc as plsc

VOCAB = 10000
EMB = 256
B = 4096
L = 200

_INFO = plsc.get_sparse_core_info()
NC = _INFO.num_cores
NS = _INFO.num_subcores
LANES = _INFO.num_lanes
NW = NC * NS
ROWS_PER_W = B // NW
G_PER_W = ROWS_PER_W // LANES
IDX_PER_W = ROWS_PER_W * L

_PROJ_BLOCK = 4096


def _proj_body(emb_ref, w_ref, out_ref):
    out_ref[:] = jnp.sum(emb_ref[:] * w_ref[:], axis=1) * (1.0 / L)


_proj_call = pl.pallas_call(
    _proj_body,
    grid=(pl.cdiv(VOCAB, _PROJ_BLOCK),),
    in_specs=[
        pl.BlockSpec((_PROJ_BLOCK, EMB), lambda i: (i, 0)),
        pl.BlockSpec((1, EMB), lambda i: (0, 0)),
    ],
    out_specs=pl.BlockSpec((_PROJ_BLOCK,), lambda i: (i,)),
    out_shape=jax.ShapeDtypeStruct((VOCAB,), jnp.float32),
)


def _sc_body(proj_hbm, resp_hbm, b_hbm, out_hbm, proj_v, resp_v, b_v, out_v, sem):
    wid = lax.axis_index("s") * NC + lax.axis_index("c")
    cp1 = pltpu.async_copy(proj_hbm, proj_v, sem)
    cp2 = pltpu.async_copy(
        resp_hbm.at[pl.ds(wid * IDX_PER_W, IDX_PER_W)], resp_v, sem
    )
    cp3 = pltpu.async_copy(b_hbm, b_v, sem)
    cp1.wait()
    cp2.wait()
    cp3.wait()
    bvec = b_v[...]
    lane = lax.iota(jnp.int32, LANES)
    svecs = [(g * LANES + lane) * L for g in range(G_PER_W)]

    def step(l, accs):
        new = []
        for g in range(G_PER_W):
            tok = plsc.load_gather(resp_v, [svecs[g] + l])
            new.append(accs[g] + plsc.load_gather(proj_v, [tok]))
        return tuple(new)

    def body(i, accs):
        return step(2 * i + 1, step(2 * i, accs))

    accs = lax.fori_loop(
        0, L // 2, body,
        tuple(jnp.zeros((LANES,), jnp.float32) for _ in range(G_PER_W)),
    )
    for g in range(G_PER_W):
        out_v[pl.ds(g * LANES, LANES)] = accs[g] + bvec
    pltpu.sync_copy(out_v, out_hbm.at[pl.ds(wid * ROWS_PER_W, ROWS_PER_W)])


_sc_call = pl.kernel(
    _sc_body,
    out_type=jax.ShapeDtypeStruct((B,), jnp.float32),
    mesh=plsc.VectorSubcoreMesh(core_axis_name="c", subcore_axis_name="s"),
    compiler_params=pltpu.CompilerParams(needs_layout_passes=False),
    scratch_types=[
        pltpu.VMEM((VOCAB,), jnp.float32),
        pltpu.VMEM((IDX_PER_W,), jnp.int32),
        pltpu.VMEM((LANES,), jnp.float32),
        pltpu.VMEM((ROWS_PER_W,), jnp.float32),
        pltpu.SemaphoreType.DMA,
    ],
)


@jax.jit
def kernel(response, emb_table, W, b):
    proj = _proj_call(emb_table, W)
    resp = response.reshape(B * L)
    b16 = jnp.broadcast_to(b, (LANES,)).astype(jnp.float32)
    out = _sc_call(proj, resp, b16)
    return out.reshape(B, 1)


# speedup vs baseline: 1.0745x; 1.0745x over previous
import jax
import jax.numpy as jnp
from jax import lax
from jax.experimental import pallas as pl
from jax.experimental.pallas import tpu as pltpu
from jax.experimental.pallas import tpu_sc as plsc

VOCAB = 10000
EMB = 256
B = 4096
L = 200

_INFO = plsc.get_sparse_core_info()
NC = _INFO.num_cores
NS = _INFO.num_subcores
LANES = _INFO.num_lanes
NW = NC * NS
ROWS_PER_W = B // NW
G_PER_W = ROWS_PER_W // LANES
IDX_PER_W = ROWS_PER_W * L

_PROJ_BLOCK = 4096


def _proj_body(emb_ref, w_ref, out_ref):
    out_ref[:] = jax.lax.dot_general(
        emb_ref[:], w_ref[0] * (1.0 / L), (((1,), (0,)), ((), ())),
        preferred_element_type=jnp.float32,
    )


_proj_call = pl.pallas_call(
    _proj_body,
    grid=(pl.cdiv(VOCAB, _PROJ_BLOCK),),
    in_specs=[
        pl.BlockSpec((_PROJ_BLOCK, EMB), lambda i: (i, 0)),
        pl.BlockSpec((1, EMB), lambda i: (0, 0)),
    ],
    out_specs=pl.BlockSpec((_PROJ_BLOCK,), lambda i: (i,)),
    out_shape=jax.ShapeDtypeStruct((VOCAB,), jnp.float32),
)


def _sc_body(proj_hbm, resp_hbm, b_hbm, out_hbm, proj_v, resp_v, b_v, out_v, sem):
    wid = lax.axis_index("s") * NC + lax.axis_index("c")
    cp1 = pltpu.async_copy(proj_hbm, proj_v, sem)
    cp2 = pltpu.async_copy(resp_hbm.at[wid], resp_v, sem)
    cp3 = pltpu.async_copy(b_hbm, b_v, sem)
    cp1.wait()
    cp2.wait()
    cp3.wait()
    bvec = b_v[...]
    lane = lax.iota(jnp.int32, LANES)
    svecs = [(g * LANES + lane) * L for g in range(G_PER_W)]

    def step(l, accs):
        new = []
        for g in range(G_PER_W):
            tok = plsc.load_gather(resp_v, [svecs[g] + l])
            new.append(accs[g] + plsc.load_gather(proj_v, [tok]))
        return tuple(new)

    def body(i, accs):
        return step(2 * i + 1, step(2 * i, accs))

    accs = lax.fori_loop(
        0, L // 2, body,
        tuple(jnp.zeros((LANES,), jnp.float32) for _ in range(G_PER_W)),
    )
    for g in range(G_PER_W):
        out_v[pl.ds(g * LANES, LANES)] = accs[g] + bvec
    pltpu.sync_copy(out_v, out_hbm.at[pl.ds(wid * ROWS_PER_W, ROWS_PER_W)])


_sc_call = pl.kernel(
    _sc_body,
    out_type=jax.ShapeDtypeStruct((B,), jnp.float32),
    mesh=plsc.VectorSubcoreMesh(core_axis_name="c", subcore_axis_name="s"),
    compiler_params=pltpu.CompilerParams(needs_layout_passes=False),
    scratch_types=[
        pltpu.VMEM((VOCAB,), jnp.float32),
        pltpu.VMEM((IDX_PER_W,), jnp.int32),
        pltpu.VMEM((LANES,), jnp.float32),
        pltpu.VMEM((ROWS_PER_W,), jnp.float32),
        pltpu.SemaphoreType.DMA,
    ],
)


@jax.jit
def kernel(response, emb_table, W, b):
    proj = _proj_call(emb_table, W)
    resp = response.reshape(NW, IDX_PER_W)
    b16 = jnp.broadcast_to(b, (LANES,)).astype(jnp.float32)
    out = _sc_call(proj, resp, b16)
    return out.reshape(B, 1)


# speedup vs baseline: 1.1011x; 1.0248x over previous
import jax
import jax.numpy as jnp
from jax import lax
from jax.experimental import pallas as pl
from jax.experimental.pallas import tpu as pltpu
from jax.experimental.pallas import tpu_sc as plsc

VOCAB = 10000
EMB = 256
B = 4096
L = 200

_INFO = plsc.get_sparse_core_info()
NC = _INFO.num_cores
NS = _INFO.num_subcores
LANES = _INFO.num_lanes
NW = NC * NS
ROWS_PER_W = B // NW
_NFULL = L // LANES
_NTAIL = L - _NFULL * LANES
_ROWS_PER_STEP = 2

_PROJ_BLOCK = 4096


def _proj_body(emb_ref, w_ref, out_ref):
    out_ref[:] = jax.lax.dot_general(
        emb_ref[:], w_ref[0] * (1.0 / L), (((1,), (0,)), ((), ())),
        preferred_element_type=jnp.float32,
    )


_proj_call = pl.pallas_call(
    _proj_body,
    grid=(pl.cdiv(VOCAB, _PROJ_BLOCK),),
    in_specs=[
        pl.BlockSpec((_PROJ_BLOCK, EMB), lambda i: (i, 0)),
        pl.BlockSpec((1, EMB), lambda i: (0, 0)),
    ],
    out_specs=pl.BlockSpec((_PROJ_BLOCK,), lambda i: (i,)),
    out_shape=jax.ShapeDtypeStruct((VOCAB,), jnp.float32),
)


def _sc_body(proj_hbm, resp_hbm, b_hbm, out_hbm, proj_v, resp_v, b_v, out_v, sem):
    wid = lax.axis_index("s") * NC + lax.axis_index("c")
    cp1 = pltpu.async_copy(proj_hbm, proj_v, sem)
    cp2 = pltpu.async_copy(
        resp_hbm.at[pl.ds(wid * ROWS_PER_W, ROWS_PER_W)], resp_v, sem
    )
    cp3 = pltpu.async_copy(b_hbm, b_v, sem)
    cp1.wait()
    cp2.wait()
    cp3.wait()
    b0 = b_v[...][0]
    lane = lax.iota(jnp.int32, LANES)
    tail_keep = lane >= (LANES - _NTAIL)
    zero = jnp.zeros((LANES,), jnp.float32)
    lane0 = lane == 0

    def row_sum(r):
        acc = zero
        for k in range(_NFULL):
            tok = resp_v[r, pl.ds(k * LANES, LANES)]
            acc = acc + plsc.load_gather(proj_v, [tok])
        tok = resp_v[r, pl.ds(L - LANES, LANES)]
        acc = acc + jnp.where(tail_keep, plsc.load_gather(proj_v, [tok]), zero)
        return jnp.sum(acc) + b0

    def body(i, carry):
        for j in range(_ROWS_PER_STEP):
            r = i * _ROWS_PER_STEP + j
            total = jnp.broadcast_to(row_sum(r), (LANES,))
            plsc.store_scatter(
                out_v, [jnp.broadcast_to(r, (LANES,))], total, mask=lane0
            )
        return carry

    lax.fori_loop(0, ROWS_PER_W // _ROWS_PER_STEP, body, 0)
    pltpu.sync_copy(out_v, out_hbm.at[pl.ds(wid * ROWS_PER_W, ROWS_PER_W)])


_sc_call = pl.kernel(
    _sc_body,
    out_type=jax.ShapeDtypeStruct((B,), jnp.float32),
    mesh=plsc.VectorSubcoreMesh(core_axis_name="c", subcore_axis_name="s"),
    compiler_params=pltpu.CompilerParams(needs_layout_passes=False),
    scratch_types=[
        pltpu.VMEM((VOCAB,), jnp.float32),
        pltpu.VMEM((ROWS_PER_W, L), jnp.int32),
        pltpu.VMEM((LANES,), jnp.float32),
        pltpu.VMEM((ROWS_PER_W,), jnp.float32),
        pltpu.SemaphoreType.DMA,
    ],
)


@jax.jit
def kernel(response, emb_table, W, b):
    proj = _proj_call(emb_table, W)
    b16 = jnp.broadcast_to(b, (LANES,)).astype(jnp.float32)
    out = _sc_call(proj, response, b16)
    return out.reshape(B, 1)


# speedup vs baseline: 1.4047x; 1.2757x over previous
import jax
import jax.numpy as jnp
from jax import lax
from jax.experimental import pallas as pl
from jax.experimental.pallas import tpu as pltpu
from jax.experimental.pallas import tpu_sc as plsc

VOCAB = 10000
EMB = 256
B = 4096
L = 200

_INFO = plsc.get_sparse_core_info()
NC = _INFO.num_cores
NS = _INFO.num_subcores
LANES = _INFO.num_lanes
NW = NC * NS
ROWS_PER_W = B // NW
G_PER_W = ROWS_PER_W // LANES

_PROJ_BLOCK = 4096


def _proj_body(emb_ref, w_ref, out_ref):
    out_ref[:] = jax.lax.dot_general(
        emb_ref[:], w_ref[0] * (1.0 / L), (((1,), (0,)), ((), ())),
        preferred_element_type=jnp.float32,
    )


_proj_call = pl.pallas_call(
    _proj_body,
    grid=(pl.cdiv(VOCAB, _PROJ_BLOCK),),
    in_specs=[
        pl.BlockSpec((_PROJ_BLOCK, EMB), lambda i: (i, 0)),
        pl.BlockSpec((1, EMB), lambda i: (0, 0)),
    ],
    out_specs=pl.BlockSpec((_PROJ_BLOCK,), lambda i: (i,)),
    out_shape=jax.ShapeDtypeStruct((VOCAB,), jnp.float32),
)


def _sc_body(proj_hbm, resp_hbm, b_hbm, out_hbm, proj_v, resp_v, b_v, out_v, sem):
    wid = lax.axis_index("s") * NC + lax.axis_index("c")
    cp1 = pltpu.async_copy(proj_hbm, proj_v, sem)
    cp2 = pltpu.async_copy(
        resp_hbm.at[:, pl.ds(wid * ROWS_PER_W, ROWS_PER_W)], resp_v, sem
    )
    cp3 = pltpu.async_copy(b_hbm, b_v, sem)
    cp1.wait()
    cp2.wait()
    cp3.wait()
    bvec = b_v[...]

    def body(l, accs):
        new = []
        for g in range(G_PER_W):
            tok = resp_v[l, pl.ds(g * LANES, LANES)]
            new.append(accs[g] + plsc.load_gather(proj_v, [tok]))
        return tuple(new)

    accs = lax.fori_loop(
        0, L, body, tuple(jnp.zeros((LANES,), jnp.float32) for _ in range(G_PER_W))
    )
    for g in range(G_PER_W):
        out_v[pl.ds(g * LANES, LANES)] = accs[g] + bvec
    pltpu.sync_copy(out_v, out_hbm.at[pl.ds(wid * ROWS_PER_W, ROWS_PER_W)])


_sc_call = pl.kernel(
    _sc_body,
    out_type=jax.ShapeDtypeStruct((B,), jnp.float32),
    mesh=plsc.VectorSubcoreMesh(core_axis_name="c", subcore_axis_name="s"),
    compiler_params=pltpu.CompilerParams(needs_layout_passes=False),
    scratch_types=[
        pltpu.VMEM((VOCAB,), jnp.float32),
        pltpu.VMEM((L, ROWS_PER_W), jnp.int32),
        pltpu.VMEM((LANES,), jnp.float32),
        pltpu.VMEM((ROWS_PER_W,), jnp.float32),
        pltpu.SemaphoreType.DMA,
    ],
)


@jax.jit
def kernel(response, emb_table, W, b):
    proj = _proj_call(emb_table, W)
    b16 = jnp.broadcast_to(b, (LANES,)).astype(jnp.float32)
    out = _sc_call(proj, response.T, b16)
    return out.reshape(B, 1)


# speedup vs baseline: 1.4430x; 1.0273x over previous
import jax
import jax.numpy as jnp
from jax import lax
from jax.experimental import pallas as pl
from jax.experimental.pallas import tpu as pltpu
from jax.experimental.pallas import tpu_sc as plsc

VOCAB = 10000
EMB = 256
B = 4096
L = 200

_INFO = plsc.get_sparse_core_info()
NC = _INFO.num_cores
NS = _INFO.num_subcores
LANES = _INFO.num_lanes
NW = NC * NS
ROWS_PER_W = B // NW
G_PER_W = ROWS_PER_W // LANES

_PROJ_CHUNK = 2000
_PROJ_NCHUNK = VOCAB // _PROJ_CHUNK


def _proj_body(emb_hbm, w_ref, out_ref, buf0, buf1, sem0, sem1):
    w = w_ref[0] * (1.0 / L)
    bufs = (buf0, buf1)
    sems = (sem0, sem1)
    cps = [None] * _PROJ_NCHUNK
    cps[0] = pltpu.async_copy(emb_hbm.at[pl.ds(0, _PROJ_CHUNK)], buf0, sem0)
    for i in range(_PROJ_NCHUNK):
        if i + 1 < _PROJ_NCHUNK:
            cps[i + 1] = pltpu.async_copy(
                emb_hbm.at[pl.ds((i + 1) * _PROJ_CHUNK, _PROJ_CHUNK)],
                bufs[(i + 1) % 2],
                sems[(i + 1) % 2],
            )
        cps[i].wait()
        out_ref[pl.ds(i * _PROJ_CHUNK, _PROJ_CHUNK)] = jax.lax.dot_general(
            bufs[i % 2][...], w, (((1,), (0,)), ((), ())),
            preferred_element_type=jnp.float32,
        )


_proj_call = pl.pallas_call(
    _proj_body,
    in_specs=[
        pl.BlockSpec(memory_space=pltpu.HBM),
        pl.BlockSpec((1, EMB), lambda: (0, 0)),
    ],
    out_specs=pl.BlockSpec((VOCAB,), lambda: (0,)),
    out_shape=jax.ShapeDtypeStruct((VOCAB,), jnp.float32),
    scratch_shapes=[
        pltpu.VMEM((_PROJ_CHUNK, EMB), jnp.float32),
        pltpu.VMEM((_PROJ_CHUNK, EMB), jnp.float32),
        pltpu.SemaphoreType.DMA,
        pltpu.SemaphoreType.DMA,
    ],
)


def _sc_body(proj_hbm, resp_hbm, b_hbm, out_hbm, proj_v, resp_v, b_v, out_v, sem):
    wid = lax.axis_index("s") * NC + lax.axis_index("c")
    cp1 = pltpu.async_copy(proj_hbm, proj_v, sem)
    cp2 = pltpu.async_copy(
        resp_hbm.at[:, pl.ds(wid * ROWS_PER_W, ROWS_PER_W)], resp_v, sem
    )
    cp3 = pltpu.async_copy(b_hbm, b_v, sem)
    cp1.wait()
    cp2.wait()
    cp3.wait()
    bvec = b_v[...]

    def body(l, accs):
        new = []
        for g in range(G_PER_W):
            tok = resp_v[l, pl.ds(g * LANES, LANES)]
            new.append(accs[g] + plsc.load_gather(proj_v, [tok]))
        return tuple(new)

    accs = lax.fori_loop(
        0, L, body, tuple(jnp.zeros((LANES,), jnp.float32) for _ in range(G_PER_W))
    )
    for g in range(G_PER_W):
        out_v[pl.ds(g * LANES, LANES)] = accs[g] + bvec
    pltpu.sync_copy(out_v, out_hbm.at[pl.ds(wid * ROWS_PER_W, ROWS_PER_W)])


_sc_call = pl.kernel(
    _sc_body,
    out_type=jax.ShapeDtypeStruct((B,), jnp.float32),
    mesh=plsc.VectorSubcoreMesh(core_axis_name="c", subcore_axis_name="s"),
    compiler_params=pltpu.CompilerParams(needs_layout_passes=False),
    scratch_types=[
        pltpu.VMEM((VOCAB,), jnp.float32),
        pltpu.VMEM((L, ROWS_PER_W), jnp.int32),
        pltpu.VMEM((LANES,), jnp.float32),
        pltpu.VMEM((ROWS_PER_W,), jnp.float32),
        pltpu.SemaphoreType.DMA,
    ],
)


@jax.jit
def kernel(response, emb_table, W, b):
    proj = _proj_call(emb_table, W)
    b16 = jnp.broadcast_to(b, (LANES,)).astype(jnp.float32)
    out = _sc_call(proj, response.T, b16)
    return out.reshape(B, 1)
